# baseline (speedup 1.0000x reference)
import functools

import jax
import jax.numpy as jnp
from jax import lax
from jax.experimental import pallas as pl
from jax.experimental.pallas import tpu as pltpu
from jax.experimental.pallas import tpu_sc as plsc

N_USERS = 10000
N_ENT = 10000
N_ITEMS = 5000
CH = 128
E = 320000
NNZ = 200000
TEMP = 0.2
N_HOPS = 2

NC = 2
NS = 16
NW = NC * NS
EB = 128
E_P = 327680
NB_E = E_P // EB // NW
NNZ_P = 204800
NB_N = NNZ_P // EB // NW
N_ACC = 10240
NROW = NNZ_P // CH

_MESH = plsc.VectorSubcoreMesh(core_axis_name="c", subcore_axis_name="s")
_f32 = jnp.float32
_i32 = jnp.int32


@functools.partial(
    pl.kernel, mesh=_MESH,
    out_type=jax.ShapeDtypeStruct((NC, N_ACC, CH), _f32),
    scratch_types=[
        pltpu.VMEM((8, EB), _i32),
        pltpu.VMEM((8, EB), _i32),
        pltpu.VMEM((EB, CH), _f32),
        pltpu.VMEM((EB, CH), _f32),
        pltpu.VMEM_SHARED((N_ACC, CH), _f32),
        pltpu.SemaphoreType.DMA,
        pltpu.SemaphoreType.DMA,
    ])
def _sc_edge(ee_hbm, tail3_hbm, head3_hbm, zeros_hbm, out_rows,
             tails_c, heads_c, r0, r1, acc, m0, m1):
    c = lax.axis_index("c")
    s = lax.axis_index("s")
    w = s * NC + c

    @pl.when(s == 0)
    def _():
        pltpu.sync_copy(zeros_hbm, acc)

    plsc.subcore_barrier()
    bufs = (r0, r1)
    sems = (m0, m1)

    def body(j, carry):
        pltpu.sync_copy(tail3_hbm.at[w, pl.ds(j * 8, 8)], tails_c)
        pltpu.sync_copy(head3_hbm.at[w, pl.ds(j * 8, 8)], heads_c)
        h0 = pltpu.async_copy(ee_hbm.at[tails_c.at[0]], r0, m0)
        h1 = pltpu.async_copy(ee_hbm.at[tails_c.at[1]], r1, m1)
        for k in range(8):
            p = k & 1
            (h0 if p == 0 else h1).wait()
            pltpu.sync_copy(bufs[p], acc.at[heads_c.at[k]], add=True)
            if k + 2 < 8:
                h = pltpu.async_copy(ee_hbm.at[tails_c.at[k + 2]],
                                     bufs[p], sems[p])
                if p == 0:
                    h0 = h
                else:
                    h1 = h
        return carry

    lax.fori_loop(0, NB_E // 8, body, 0)
    plsc.subcore_barrier()

    @pl.when(s == 0)
    def _():
        pltpu.sync_copy(acc, out_rows.at[c])


@functools.partial(
    pl.kernel, mesh=_MESH,
    out_type=jax.ShapeDtypeStruct((NC, N_ACC, CH), _f32),
    scratch_types=[
        pltpu.VMEM((NB_E, EB), _i32),
        pltpu.VMEM((EB, CH), _f32),
        pltpu.VMEM_SHARED((N_ACC, CH), _f32),
        pltpu.SemaphoreType.DMA,
        pltpu.SemaphoreType.DMA,
        pltpu.SemaphoreType.DMA,
        pltpu.SemaphoreType.DMA,
    ])
def _sc_cnt(head3_hbm, zeros_hbm, ones_hbm, out_cnt,
            heads_v, ones_v, cacc, m0, m1, m2, m3):
    c = lax.axis_index("c")
    s = lax.axis_index("s")
    w = s * NC + c

    @pl.when(s == 0)
    def _():
        pltpu.sync_copy(zeros_hbm, cacc)

    def fill(i, carry):
        pltpu.sync_copy(ones_hbm, ones_v.at[i])
        return carry

    lax.fori_loop(0, EB, fill, 0)
    pltpu.sync_copy(head3_hbm.at[w], heads_v)
    plsc.subcore_barrier()

    sems = (m0, m1, m2, m3)

    def body(j, carry):
        i0 = j * 4
        hs = [pltpu.async_copy(ones_v, cacc.at[heads_v.at[i0 + k]],
                               sems[k], add=True) for k in range(4)]
        for k in range(4):
            hs[k].wait()
        return carry

    lax.fori_loop(0, NB_E // 4, body, 0)
    plsc.subcore_barrier()

    @pl.when(s == 0)
    def _():
        pltpu.sync_copy(cacc, out_cnt.at[c])


@functools.partial(
    pl.kernel, mesh=_MESH,
    out_type=[jax.ShapeDtypeStruct((NNZ_P, CH), _f32),
              jax.ShapeDtypeStruct((NC, N_ACC, CH), _f32)],
    scratch_types=[
        pltpu.VMEM((NB_N, EB), _i32),
        pltpu.VMEM((NB_N, EB), _i32),
        pltpu.VMEM((EB, CH), _f32),
        pltpu.VMEM((EB, CH), _f32),
        pltpu.VMEM_SHARED((N_ACC, CH), _f32),
        pltpu.SemaphoreType.DMA,
        pltpu.SemaphoreType.DMA,
    ])
def _sc_gather_scatter(tab_hbm, idx3_hbm, u3_hbm, zeros_hbm,
                       out_rows, out_part,
                       idxs_v, us_v, r0, r1, acc, m0, m1):
    c = lax.axis_index("c")
    s = lax.axis_index("s")
    w = s * NC + c

    @pl.when(s == 0)
    def _():
        pltpu.sync_copy(zeros_hbm, acc)

    pltpu.sync_copy(idx3_hbm.at[w], idxs_v)
    pltpu.sync_copy(u3_hbm.at[w], us_v)
    plsc.subcore_barrier()

    def consume(buf, i):
        off = (w * NB_N + i) * EB
        pltpu.sync_copy(buf, out_rows.at[pl.ds(off, EB)])
        pltpu.sync_copy(buf, acc.at[us_v.at[i]], add=True)

    def body(j, carry):
        i0 = j * 2
        h0 = pltpu.async_copy(tab_hbm.at[idxs_v.at[i0]], r0, m0)
        h1 = pltpu.async_copy(tab_hbm.at[idxs_v.at[i0 + 1]], r1, m1)
        h0.wait()
        consume(r0, i0)
        h1.wait()
        consume(r1, i0 + 1)
        return carry

    lax.fori_loop(0, NB_N // 2, body, 0)
    plsc.subcore_barrier()

    @pl.when(s == 0)
    def _():
        pltpu.sync_copy(acc, out_part.at[c])


@functools.partial(
    pl.kernel, mesh=_MESH,
    out_type=jax.ShapeDtypeStruct((NNZ_P, CH), _f32),
    scratch_types=[
        pltpu.VMEM((NB_N, EB), _i32),
        pltpu.VMEM((EB, CH), _f32),
        pltpu.VMEM((EB, CH), _f32),
        pltpu.VMEM((EB, CH), _f32),
        pltpu.VMEM((EB, CH), _f32),
        pltpu.SemaphoreType.DMA,
        pltpu.SemaphoreType.DMA,
        pltpu.SemaphoreType.DMA,
        pltpu.SemaphoreType.DMA,
    ])
def _sc_gather(tab_hbm, u3_hbm, out_rows,
               us_v, r0, r1, r2, r3, m0, m1, m2, m3):
    c = lax.axis_index("c")
    s = lax.axis_index("s")
    w = s * NC + c
    pltpu.sync_copy(u3_hbm.at[w], us_v)

    bufs = (r0, r1, r2, r3)
    sems = (m0, m1, m2, m3)

    def run(i0, nk):
        hs = [pltpu.async_copy(tab_hbm.at[us_v.at[i0 + k]], bufs[k],
                               sems[k]) for k in range(nk)]
        for k in range(nk):
            hs[k].wait()
            off = (w * NB_N + i0 + k) * EB
            pltpu.sync_copy(bufs[k], out_rows.at[pl.ds(off, EB)])

    def body(j, carry):
        run(j * 4, 4)
        return carry

    lax.fori_loop(0, NB_N // 4, body, 0)
    run((NB_N // 4) * 4, NB_N % 4)


@functools.partial(
    pl.kernel, mesh=_MESH,
    out_type=jax.ShapeDtypeStruct((NC, N_ACC, CH), _f32),
    scratch_types=[
        pltpu.VMEM((NB_N, EB), _i32),
        pltpu.VMEM((EB, CH), _f32),
        pltpu.VMEM((EB, CH), _f32),
        pltpu.VMEM_SHARED((N_ACC, CH), _f32),
        pltpu.SemaphoreType.DMA,
        pltpu.SemaphoreType.DMA,
    ])
def _sc_scatter(w_hbm, u3_hbm, zeros_hbm, out_part,
                us_v, r0, r1, acc, m0, m1):
    c = lax.axis_index("c")
    s = lax.axis_index("s")
    w = s * NC + c

    @pl.when(s == 0)
    def _():
        pltpu.sync_copy(zeros_hbm, acc)

    pltpu.sync_copy(u3_hbm.at[w], us_v)
    plsc.subcore_barrier()

    def body(j, carry):
        i0 = j * 2
        off0 = (w * NB_N + i0) * EB
        h0 = pltpu.async_copy(w_hbm.at[pl.ds(off0, EB)], r0, m0)
        h1 = pltpu.async_copy(w_hbm.at[pl.ds(off0 + EB, EB)], r1, m1)
        h0.wait()
        pltpu.sync_copy(r0, acc.at[us_v.at[i0]], add=True)
        h1.wait()
        pltpu.sync_copy(r1, acc.at[us_v.at[i0 + 1]], add=True)
        return carry

    lax.fori_loop(0, NB_N // 2, body, 0)
    plsc.subcore_barrier()

    @pl.when(s == 0)
    def _():
        pltpu.sync_copy(acc, out_part.at[c])


_BR = 512
_NBLK = N_ACC // _BR


def _tc_entity_body(p_ref, c_ref, res_ref, agg_ref, ee_ref, out_ref):
    sums = p_ref[0] + p_ref[1]
    cnt = jnp.maximum(c_ref[0, :, 0:1] + c_ref[1, :, 0:1], 1.0)
    agg = sums / cnt
    ss = jnp.sum(agg * agg, axis=1, keepdims=True)
    nn = agg * lax.rsqrt(jnp.maximum(ss, 1e-24))
    agg_ref[...] = agg
    ee_ref[...] = nn
    out_ref[...] = res_ref[...] + nn


def _tc_entity(p, cnt_p, res):
    return pl.pallas_call(
        _tc_entity_body,
        grid=(_NBLK,),
        in_specs=[
            pl.BlockSpec((NC, _BR, CH), lambda i: (0, i, 0)),
            pl.BlockSpec((NC, _BR, CH), lambda i: (0, i, 0)),
            pl.BlockSpec((_BR, CH), lambda i: (i, 0)),
        ],
        out_specs=[
            pl.BlockSpec((_BR, CH), lambda i: (i, 0)),
            pl.BlockSpec((_BR, CH), lambda i: (i, 0)),
            pl.BlockSpec((_BR, CH), lambda i: (i, 0)),
        ],
        out_shape=[
            jax.ShapeDtypeStruct((N_ACC, CH), _f32),
            jax.ShapeDtypeStruct((N_ACC, CH), _f32),
            jax.ShapeDtypeStruct((N_ACC, CH), _f32),
        ],
    )(p, cnt_p, res)


def _tc_user_body(p_ref, res_ref, out_ref):
    agg = p_ref[0] + p_ref[1]
    ss = jnp.sum(agg * agg, axis=1, keepdims=True)
    nn = agg * lax.rsqrt(jnp.maximum(ss, 1e-24))
    out_ref[...] = res_ref[...] + nn


def _tc_user(p, res):
    return pl.pallas_call(
        _tc_user_body,
        grid=(_NBLK,),
        in_specs=[
            pl.BlockSpec((NC, _BR, CH), lambda i: (0, i, 0)),
            pl.BlockSpec((_BR, CH), lambda i: (i, 0)),
        ],
        out_specs=pl.BlockSpec((_BR, CH), lambda i: (i, 0)),
        out_shape=jax.ShapeDtypeStruct((N_ACC, CH), _f32),
    )(p, res)


def _tc_combine_body(p_ref, out_ref):
    out_ref[...] = p_ref[0] + p_ref[1]


def _tc_combine(p):
    return pl.pallas_call(
        _tc_combine_body,
        grid=(_NBLK,),
        in_specs=[pl.BlockSpec((NC, _BR, CH), lambda i: (0, i, 0))],
        out_specs=pl.BlockSpec((_BR, CH), lambda i: (i, 0)),
        out_shape=jax.ShapeDtypeStruct((N_ACC, CH), _f32),
    )(p)


_SB = 8


def _tc_score_body(r_ref, m_ref, s_ref):
    d = r_ref[...] - m_ref[...] + 1e-6
    s_ref[...] = jnp.sum(d * d, axis=2)


def _tc_score(r3, m3):
    return pl.pallas_call(
        _tc_score_body,
        grid=(NROW // _SB,),
        in_specs=[
            pl.BlockSpec((_SB, CH, CH), lambda i: (i, 0, 0)),
            pl.BlockSpec((_SB, CH, CH), lambda i: (i, 0, 0)),
        ],
        out_specs=pl.BlockSpec((_SB, CH), lambda i: (i, 0)),
        out_shape=jax.ShapeDtypeStruct((NROW, CH), _f32),
    )(r3, m3)


_KSTEPS = (1, 2, 4, 8, 16, 32, 64, 128, 256, 512, 1024)


def _shift_dn(x, k, fill):
    rr, cc = x.shape
    if k % cc == 0:
        r = k // cc
        top = jnp.full((r, cc), fill, x.dtype)
        return jnp.concatenate([top, x[:-r]], axis=0)
    xprev = jnp.concatenate(
        [jnp.full((1, cc), fill, x.dtype), x[:-1]], axis=0)
    return jnp.concatenate([xprev[:, cc - k:], x[:, :cc - k]], axis=1)


def _shift_up(x, k, fill):
    rr, cc = x.shape
    if k % cc == 0:
        r = k // cc
        bot = jnp.full((r, cc), fill, x.dtype)
        return jnp.concatenate([x[r:], bot], axis=0)
    xnext = jnp.concatenate(
        [x[1:], jnp.full((1, cc), fill, x.dtype)], axis=0)
    return jnp.concatenate([x[:, k:], xnext[:, :k]], axis=1)


def _tc_softmax_body(s2_ref, u_ref, soft_ref):
    ss = s2_ref[...]
    u = u_ref[...]
    s = jnp.sqrt(ss) * (1.0 / TEMP)
    m = s
    for k in _KSTEPS:
        us = _shift_dn(u, k, -1)
        ms = _shift_dn(m, k, 0.0)
        m = jnp.where(us == u, jnp.maximum(m, ms), m)
    mb = s
    for k in _KSTEPS:
        us = _shift_up(u, k, -1)
        ms = _shift_up(mb, k, 0.0)
        mb = jnp.where(us == u, jnp.maximum(mb, ms), mb)
    mm = jnp.maximum(m, mb)
    e = jnp.exp(s - mm)
    lf = e
    for k in _KSTEPS:
        us = _shift_dn(u, k, -1)
        ls = _shift_dn(lf, k, 0.0)
        lf = lf + jnp.where(us == u, ls, 0.0)
    lb = e
    for k in _KSTEPS:
        us = _shift_up(u, k, -1)
        ls = _shift_up(lb, k, 0.0)
        lb = lb + jnp.where(us == u, ls, 0.0)
    denom = lf + lb - e
    soft_ref[...] = e / denom


def _tc_softmax(s2, u2):
    return pl.pallas_call(
        _tc_softmax_body,
        out_shape=jax.ShapeDtypeStruct((NROW, CH), _f32),
    )(s2, u2)


def _tc_wmul_body(soft_ref, r_ref, w_ref):
    w_ref[...] = r_ref[...] * soft_ref[...][:, :, None]


def _tc_wmul(soft2, r3):
    return pl.pallas_call(
        _tc_wmul_body,
        grid=(NROW // _SB,),
        in_specs=[
            pl.BlockSpec((_SB, CH), lambda i: (i, 0)),
            pl.BlockSpec((_SB, CH, CH), lambda i: (i, 0, 0)),
        ],
        out_specs=pl.BlockSpec((_SB, CH, CH), lambda i: (i, 0, 0)),
        out_shape=jax.ShapeDtypeStruct((NROW, CH, CH), _f32),
    )(soft2, r3)


def kernel(user_emb, entity_emb, weight, interact_values, edge_index,
           edge_type, interact_user, interact_item):
    tail3 = jnp.concatenate(
        [edge_index[1].astype(_i32), jnp.zeros((E_P - E,), _i32)]
    ).reshape(NW, NB_E, EB)
    head3 = jnp.concatenate(
        [edge_index[0].astype(_i32), jnp.full((E_P - E,), N_ENT, _i32)]
    ).reshape(NW, NB_E, EB)
    i3 = jnp.concatenate(
        [interact_item.astype(_i32), jnp.zeros((NNZ_P - NNZ,), _i32)]
    ).reshape(NW, NB_N, EB)
    u_p = jnp.concatenate(
        [interact_user.astype(_i32), jnp.full((NNZ_P - NNZ,), N_USERS, _i32)])
    u3 = u_p.reshape(NW, NB_N, EB)
    u2 = u_p.reshape(NROW, CH)
    zeros = jnp.zeros((N_ACC, CH), _f32)
    ones_row = jnp.zeros((CH,), _f32).at[0].set(1.0)
    pad_rows = jnp.zeros((N_ACC - N_ENT, CH), _f32)

    ee = jnp.concatenate([entity_emb, pad_rows])
    ent_res = jnp.concatenate([entity_emb, pad_rows])
    usr_res = jnp.concatenate([user_emb, pad_rows])

    cnt_p = _sc_cnt(head3, zeros, ones_row)
    for _ in range(N_HOPS):
        rows_p = _sc_edge(ee, tail3, head3, zeros)
        entity_agg, ee, ent_res = _tc_entity(rows_p, cnt_p, ent_res)
        r_rows, um_p = _sc_gather_scatter(entity_agg, i3, u3, zeros)
        user_mean = _tc_combine(um_p)
        m_rows = _sc_gather(user_mean, u3)
        r3 = r_rows.reshape(NROW, CH, CH)
        s2 = _tc_score(r3, m_rows.reshape(NROW, CH, CH))
        soft2 = _tc_softmax(s2, u2)
        w3 = _tc_wmul(soft2, r3)
        ua_p = _sc_scatter(w3.reshape(NNZ_P, CH), u3, zeros)
        usr_res = _tc_user(ua_p, usr_res)

    return (usr_res[:N_USERS], ent_res[:N_ENT])


# speedup vs baseline: 1.0102x; 1.0102x over previous
import functools

import jax
import jax.numpy as jnp
from jax import lax
from jax.experimental import pallas as pl
from jax.experimental.pallas import tpu as pltpu
from jax.experimental.pallas import tpu_sc as plsc

N_USERS = 10000
N_ENT = 10000
N_ITEMS = 5000
CH = 128
E = 320000
NNZ = 200000
TEMP = 0.2
N_HOPS = 2

NC = 2
NS = 16
NW = NC * NS
EB = 128
E_P = 327680
NB_E = E_P // EB // NW
NNZ_P = 204800
NB_N = NNZ_P // EB // NW
N_ACC = 10240
NROW = NNZ_P // CH

_MESH = plsc.VectorSubcoreMesh(core_axis_name="c", subcore_axis_name="s")
_f32 = jnp.float32
_i32 = jnp.int32


@functools.partial(
    pl.kernel, mesh=_MESH,
    out_type=jax.ShapeDtypeStruct((NC, N_ACC, CH), _f32),
    scratch_types=[
        pltpu.VMEM((8, EB), _i32),
        pltpu.VMEM((8, EB), _i32),
        pltpu.VMEM((EB, CH), _f32),
        pltpu.VMEM((EB, CH), _f32),
        pltpu.VMEM_SHARED((N_ACC, CH), _f32),
        pltpu.SemaphoreType.DMA,
        pltpu.SemaphoreType.DMA,
        pltpu.SemaphoreType.DMA,
        pltpu.SemaphoreType.DMA,
    ])
def _sc_edge(ee_hbm, tail3_hbm, head3_hbm, zeros_hbm, out_rows,
             tails_c, heads_c, r0, r1, acc, m0, m1, n0, n1):
    c = lax.axis_index("c")
    s = lax.axis_index("s")
    w = s * NC + c

    @pl.when(s == 0)
    def _():
        pltpu.sync_copy(zeros_hbm, acc)

    plsc.subcore_barrier()
    bufs = (r0, r1)
    sems = (m0, m1)

    ssems = (n0, n1)

    def body(j, carry):
        pltpu.sync_copy(tail3_hbm.at[w, pl.ds(j * 8, 8)], tails_c)
        pltpu.sync_copy(head3_hbm.at[w, pl.ds(j * 8, 8)], heads_c)
        g0 = pltpu.async_copy(ee_hbm.at[tails_c.at[0]], r0, m0)
        g1 = pltpu.async_copy(ee_hbm.at[tails_c.at[1]], r1, m1)
        g0.wait()
        s0 = pltpu.async_copy(r0, acc.at[heads_c.at[0]], n0, add=True)
        g1.wait()
        s1 = pltpu.async_copy(r1, acc.at[heads_c.at[1]], n1, add=True)
        for k in range(2, 8):
            p = k & 1
            (s0 if p == 0 else s1).wait()
            g = pltpu.async_copy(ee_hbm.at[tails_c.at[k]], bufs[p], sems[p])
            g.wait()
            ns = pltpu.async_copy(bufs[p], acc.at[heads_c.at[k]],
                                  ssems[p], add=True)
            if p == 0:
                s0 = ns
            else:
                s1 = ns
        s0.wait()
        s1.wait()
        return carry

    lax.fori_loop(0, NB_E // 8, body, 0)
    plsc.subcore_barrier()

    @pl.when(s == 0)
    def _():
        pltpu.sync_copy(acc, out_rows.at[c])


@functools.partial(
    pl.kernel, mesh=_MESH,
    out_type=jax.ShapeDtypeStruct((NC, N_ACC, CH), _f32),
    scratch_types=[
        pltpu.VMEM((NB_E, EB), _i32),
        pltpu.VMEM((EB, CH), _f32),
        pltpu.VMEM_SHARED((N_ACC, CH), _f32),
        pltpu.SemaphoreType.DMA,
        pltpu.SemaphoreType.DMA,
        pltpu.SemaphoreType.DMA,
        pltpu.SemaphoreType.DMA,
    ])
def _sc_cnt(head3_hbm, zeros_hbm, ones_hbm, out_cnt,
            heads_v, ones_v, cacc, m0, m1, m2, m3):
    c = lax.axis_index("c")
    s = lax.axis_index("s")
    w = s * NC + c

    @pl.when(s == 0)
    def _():
        pltpu.sync_copy(zeros_hbm, cacc)

    def fill(i, carry):
        pltpu.sync_copy(ones_hbm, ones_v.at[i])
        return carry

    lax.fori_loop(0, EB, fill, 0)
    pltpu.sync_copy(head3_hbm.at[w], heads_v)
    plsc.subcore_barrier()

    sems = (m0, m1, m2, m3)

    def body(j, carry):
        i0 = j * 4
        hs = [pltpu.async_copy(ones_v, cacc.at[heads_v.at[i0 + k]],
                               sems[k], add=True) for k in range(4)]
        for k in range(4):
            hs[k].wait()
        return carry

    lax.fori_loop(0, NB_E // 4, body, 0)
    plsc.subcore_barrier()

    @pl.when(s == 0)
    def _():
        pltpu.sync_copy(cacc, out_cnt.at[c])


@functools.partial(
    pl.kernel, mesh=_MESH,
    out_type=[jax.ShapeDtypeStruct((NNZ_P, CH), _f32),
              jax.ShapeDtypeStruct((NC, N_ACC, CH), _f32)],
    scratch_types=[
        pltpu.VMEM((NB_N, EB), _i32),
        pltpu.VMEM((NB_N, EB), _i32),
        pltpu.VMEM((EB, CH), _f32),
        pltpu.VMEM((EB, CH), _f32),
        pltpu.VMEM_SHARED((N_ACC, CH), _f32),
        pltpu.SemaphoreType.DMA,
        pltpu.SemaphoreType.DMA,
        pltpu.SemaphoreType.DMA,
        pltpu.SemaphoreType.DMA,
        pltpu.SemaphoreType.DMA,
        pltpu.SemaphoreType.DMA,
    ])
def _sc_gather_scatter(tab_hbm, idx3_hbm, u3_hbm, zeros_hbm,
                       out_rows, out_part,
                       idxs_v, us_v, r0, r1, acc, m0, m1, w0s, w1s, n0, n1):
    c = lax.axis_index("c")
    s = lax.axis_index("s")
    w = s * NC + c

    @pl.when(s == 0)
    def _():
        pltpu.sync_copy(zeros_hbm, acc)

    pltpu.sync_copy(idx3_hbm.at[w], idxs_v)
    pltpu.sync_copy(u3_hbm.at[w], us_v)
    plsc.subcore_barrier()

    def fire(buf, i, wsem, ssem):
        off = (w * NB_N + i) * EB
        hw = pltpu.async_copy(buf, out_rows.at[pl.ds(off, EB)], wsem)
        hs = pltpu.async_copy(buf, acc.at[us_v.at[i]], ssem, add=True)
        return hw, hs

    def quad(i0):
        g0 = pltpu.async_copy(tab_hbm.at[idxs_v.at[i0]], r0, m0)
        g1 = pltpu.async_copy(tab_hbm.at[idxs_v.at[i0 + 1]], r1, m1)
        g0.wait()
        hw0, hs0 = fire(r0, i0, w0s, n0)
        g1.wait()
        hw1, hs1 = fire(r1, i0 + 1, w1s, n1)
        hw0.wait()
        hs0.wait()
        g0 = pltpu.async_copy(tab_hbm.at[idxs_v.at[i0 + 2]], r0, m0)
        hw1.wait()
        hs1.wait()
        g1 = pltpu.async_copy(tab_hbm.at[idxs_v.at[i0 + 3]], r1, m1)
        g0.wait()
        hw0, hs0 = fire(r0, i0 + 2, w0s, n0)
        g1.wait()
        hw1, hs1 = fire(r1, i0 + 3, w1s, n1)
        hw0.wait()
        hs0.wait()
        hw1.wait()
        hs1.wait()

    def body(j, carry):
        quad(j * 4)
        return carry

    lax.fori_loop(0, NB_N // 4, body, 0)
    i0 = (NB_N // 4) * 4
    g0 = pltpu.async_copy(tab_hbm.at[idxs_v.at[i0]], r0, m0)
    g1 = pltpu.async_copy(tab_hbm.at[idxs_v.at[i0 + 1]], r1, m1)
    g0.wait()
    hw0, hs0 = fire(r0, i0, w0s, n0)
    g1.wait()
    hw1, hs1 = fire(r1, i0 + 1, w1s, n1)
    hw0.wait()
    hs0.wait()
    hw1.wait()
    hs1.wait()
    plsc.subcore_barrier()

    @pl.when(s == 0)
    def _():
        pltpu.sync_copy(acc, out_part.at[c])


@functools.partial(
    pl.kernel, mesh=_MESH,
    out_type=jax.ShapeDtypeStruct((NNZ_P, CH), _f32),
    scratch_types=[
        pltpu.VMEM((NB_N, EB), _i32),
        pltpu.VMEM((EB, CH), _f32),
        pltpu.VMEM((EB, CH), _f32),
        pltpu.VMEM((EB, CH), _f32),
        pltpu.VMEM((EB, CH), _f32),
        pltpu.SemaphoreType.DMA,
        pltpu.SemaphoreType.DMA,
        pltpu.SemaphoreType.DMA,
        pltpu.SemaphoreType.DMA,
        pltpu.SemaphoreType.DMA,
        pltpu.SemaphoreType.DMA,
        pltpu.SemaphoreType.DMA,
        pltpu.SemaphoreType.DMA,
    ])
def _sc_gather(tab_hbm, u3_hbm, out_rows,
               us_v, r0, r1, r2, r3, m0, m1, m2, m3, q0, q1, q2, q3):
    c = lax.axis_index("c")
    s = lax.axis_index("s")
    w = s * NC + c
    pltpu.sync_copy(u3_hbm.at[w], us_v)

    bufs = (r0, r1, r2, r3)
    sems = (m0, m1, m2, m3)
    wsems = (q0, q1, q2, q3)

    def run(i0, nk):
        hs = [pltpu.async_copy(tab_hbm.at[us_v.at[i0 + k]], bufs[k],
                               sems[k]) for k in range(nk)]
        ws = []
        for k in range(nk):
            hs[k].wait()
            off = (w * NB_N + i0 + k) * EB
            ws.append(pltpu.async_copy(bufs[k], out_rows.at[pl.ds(off, EB)],
                                       wsems[k]))
        for k in range(nk):
            ws[k].wait()

    def body(j, carry):
        run(j * 4, 4)
        return carry

    lax.fori_loop(0, NB_N // 4, body, 0)
    run((NB_N // 4) * 4, NB_N % 4)


@functools.partial(
    pl.kernel, mesh=_MESH,
    out_type=jax.ShapeDtypeStruct((NC, N_ACC, CH), _f32),
    scratch_types=[
        pltpu.VMEM((NB_N, EB), _i32),
        pltpu.VMEM((EB, CH), _f32),
        pltpu.VMEM((EB, CH), _f32),
        pltpu.VMEM_SHARED((N_ACC, CH), _f32),
        pltpu.SemaphoreType.DMA,
        pltpu.SemaphoreType.DMA,
        pltpu.SemaphoreType.DMA,
        pltpu.SemaphoreType.DMA,
    ])
def _sc_scatter(w_hbm, u3_hbm, zeros_hbm, out_part,
                us_v, r0, r1, acc, m0, m1, n0, n1):
    c = lax.axis_index("c")
    s = lax.axis_index("s")
    w = s * NC + c

    @pl.when(s == 0)
    def _():
        pltpu.sync_copy(zeros_hbm, acc)

    pltpu.sync_copy(u3_hbm.at[w], us_v)
    plsc.subcore_barrier()

    def quad(i0):
        off0 = (w * NB_N + i0) * EB
        g0 = pltpu.async_copy(w_hbm.at[pl.ds(off0, EB)], r0, m0)
        g1 = pltpu.async_copy(w_hbm.at[pl.ds(off0 + EB, EB)], r1, m1)
        g0.wait()
        s0 = pltpu.async_copy(r0, acc.at[us_v.at[i0]], n0, add=True)
        g1.wait()
        s1 = pltpu.async_copy(r1, acc.at[us_v.at[i0 + 1]], n1, add=True)
        s0.wait()
        g0 = pltpu.async_copy(w_hbm.at[pl.ds(off0 + 2 * EB, EB)], r0, m0)
        s1.wait()
        g1 = pltpu.async_copy(w_hbm.at[pl.ds(off0 + 3 * EB, EB)], r1, m1)
        g0.wait()
        s0 = pltpu.async_copy(r0, acc.at[us_v.at[i0 + 2]], n0, add=True)
        g1.wait()
        s1 = pltpu.async_copy(r1, acc.at[us_v.at[i0 + 3]], n1, add=True)
        s0.wait()
        s1.wait()

    def body(j, carry):
        quad(j * 4)
        return carry

    lax.fori_loop(0, NB_N // 4, body, 0)
    i0 = (NB_N // 4) * 4
    off0 = (w * NB_N + i0) * EB
    g0 = pltpu.async_copy(w_hbm.at[pl.ds(off0, EB)], r0, m0)
    g1 = pltpu.async_copy(w_hbm.at[pl.ds(off0 + EB, EB)], r1, m1)
    g0.wait()
    s0 = pltpu.async_copy(r0, acc.at[us_v.at[i0]], n0, add=True)
    g1.wait()
    s1 = pltpu.async_copy(r1, acc.at[us_v.at[i0 + 1]], n1, add=True)
    s0.wait()
    s1.wait()
    plsc.subcore_barrier()

    @pl.when(s == 0)
    def _():
        pltpu.sync_copy(acc, out_part.at[c])


_BR = 512
_NBLK = N_ACC // _BR


def _tc_entity_body(p_ref, c_ref, res_ref, agg_ref, ee_ref, out_ref):
    sums = p_ref[0] + p_ref[1]
    cnt = jnp.maximum(c_ref[0, :, 0:1] + c_ref[1, :, 0:1], 1.0)
    agg = sums / cnt
    ss = jnp.sum(agg * agg, axis=1, keepdims=True)
    nn = agg * lax.rsqrt(jnp.maximum(ss, 1e-24))
    agg_ref[...] = agg
    ee_ref[...] = nn
    out_ref[...] = res_ref[...] + nn


def _tc_entity(p, cnt_p, res):
    return pl.pallas_call(
        _tc_entity_body,
        grid=(_NBLK,),
        in_specs=[
            pl.BlockSpec((NC, _BR, CH), lambda i: (0, i, 0)),
            pl.BlockSpec((NC, _BR, CH), lambda i: (0, i, 0)),
            pl.BlockSpec((_BR, CH), lambda i: (i, 0)),
        ],
        out_specs=[
            pl.BlockSpec((_BR, CH), lambda i: (i, 0)),
            pl.BlockSpec((_BR, CH), lambda i: (i, 0)),
            pl.BlockSpec((_BR, CH), lambda i: (i, 0)),
        ],
        out_shape=[
            jax.ShapeDtypeStruct((N_ACC, CH), _f32),
            jax.ShapeDtypeStruct((N_ACC, CH), _f32),
            jax.ShapeDtypeStruct((N_ACC, CH), _f32),
        ],
    )(p, cnt_p, res)


def _tc_user_body(p_ref, res_ref, out_ref):
    agg = p_ref[0] + p_ref[1]
    ss = jnp.sum(agg * agg, axis=1, keepdims=True)
    nn = agg * lax.rsqrt(jnp.maximum(ss, 1e-24))
    out_ref[...] = res_ref[...] + nn


def _tc_user(p, res):
    return pl.pallas_call(
        _tc_user_body,
        grid=(_NBLK,),
        in_specs=[
            pl.BlockSpec((NC, _BR, CH), lambda i: (0, i, 0)),
            pl.BlockSpec((_BR, CH), lambda i: (i, 0)),
        ],
        out_specs=pl.BlockSpec((_BR, CH), lambda i: (i, 0)),
        out_shape=jax.ShapeDtypeStruct((N_ACC, CH), _f32),
    )(p, res)


def _tc_combine_body(p_ref, out_ref):
    out_ref[...] = p_ref[0] + p_ref[1]


def _tc_combine(p):
    return pl.pallas_call(
        _tc_combine_body,
        grid=(_NBLK,),
        in_specs=[pl.BlockSpec((NC, _BR, CH), lambda i: (0, i, 0))],
        out_specs=pl.BlockSpec((_BR, CH), lambda i: (i, 0)),
        out_shape=jax.ShapeDtypeStruct((N_ACC, CH), _f32),
    )(p)


_SB = 8


def _tc_score_body(r_ref, m_ref, s_ref):
    d = r_ref[...] - m_ref[...] + 1e-6
    s_ref[...] = jnp.sum(d * d, axis=2)


def _tc_score(r3, m3):
    return pl.pallas_call(
        _tc_score_body,
        grid=(NROW // _SB,),
        in_specs=[
            pl.BlockSpec((_SB, CH, CH), lambda i: (i, 0, 0)),
            pl.BlockSpec((_SB, CH, CH), lambda i: (i, 0, 0)),
        ],
        out_specs=pl.BlockSpec((_SB, CH), lambda i: (i, 0)),
        out_shape=jax.ShapeDtypeStruct((NROW, CH), _f32),
    )(r3, m3)


_KSTEPS = (1, 2, 4, 8, 16, 32, 64, 128, 256, 512, 1024)


def _shift_dn(x, k, fill):
    rr, cc = x.shape
    if k % cc == 0:
        r = k // cc
        top = jnp.full((r, cc), fill, x.dtype)
        return jnp.concatenate([top, x[:-r]], axis=0)
    xprev = jnp.concatenate(
        [jnp.full((1, cc), fill, x.dtype), x[:-1]], axis=0)
    return jnp.concatenate([xprev[:, cc - k:], x[:, :cc - k]], axis=1)


def _shift_up(x, k, fill):
    rr, cc = x.shape
    if k % cc == 0:
        r = k // cc
        bot = jnp.full((r, cc), fill, x.dtype)
        return jnp.concatenate([x[r:], bot], axis=0)
    xnext = jnp.concatenate(
        [x[1:], jnp.full((1, cc), fill, x.dtype)], axis=0)
    return jnp.concatenate([x[:, k:], xnext[:, :k]], axis=1)


def _tc_softmax_body(s2_ref, u_ref, soft_ref):
    ss = s2_ref[...]
    u = u_ref[...]
    s = jnp.sqrt(ss) * (1.0 / TEMP)
    m = s
    for k in _KSTEPS:
        us = _shift_dn(u, k, -1)
        ms = _shift_dn(m, k, 0.0)
        m = jnp.where(us == u, jnp.maximum(m, ms), m)
    mb = s
    for k in _KSTEPS:
        us = _shift_up(u, k, -1)
        ms = _shift_up(mb, k, 0.0)
        mb = jnp.where(us == u, jnp.maximum(mb, ms), mb)
    mm = jnp.maximum(m, mb)
    e = jnp.exp(s - mm)
    lf = e
    for k in _KSTEPS:
        us = _shift_dn(u, k, -1)
        ls = _shift_dn(lf, k, 0.0)
        lf = lf + jnp.where(us == u, ls, 0.0)
    lb = e
    for k in _KSTEPS:
        us = _shift_up(u, k, -1)
        ls = _shift_up(lb, k, 0.0)
        lb = lb + jnp.where(us == u, ls, 0.0)
    denom = lf + lb - e
    soft_ref[...] = e / denom


def _tc_softmax(s2, u2):
    return pl.pallas_call(
        _tc_softmax_body,
        out_shape=jax.ShapeDtypeStruct((NROW, CH), _f32),
    )(s2, u2)


def _tc_wmul_body(soft_ref, r_ref, w_ref):
    w_ref[...] = r_ref[...] * soft_ref[...][:, :, None]


def _tc_wmul(soft2, r3):
    return pl.pallas_call(
        _tc_wmul_body,
        grid=(NROW // _SB,),
        in_specs=[
            pl.BlockSpec((_SB, CH), lambda i: (i, 0)),
            pl.BlockSpec((_SB, CH, CH), lambda i: (i, 0, 0)),
        ],
        out_specs=pl.BlockSpec((_SB, CH, CH), lambda i: (i, 0, 0)),
        out_shape=jax.ShapeDtypeStruct((NROW, CH, CH), _f32),
    )(soft2, r3)


def kernel(user_emb, entity_emb, weight, interact_values, edge_index,
           edge_type, interact_user, interact_item):
    tail3 = jnp.concatenate(
        [edge_index[1].astype(_i32), jnp.zeros((E_P - E,), _i32)]
    ).reshape(NW, NB_E, EB)
    head3 = jnp.concatenate(
        [edge_index[0].astype(_i32), jnp.full((E_P - E,), N_ENT, _i32)]
    ).reshape(NW, NB_E, EB)
    i3 = jnp.concatenate(
        [interact_item.astype(_i32), jnp.zeros((NNZ_P - NNZ,), _i32)]
    ).reshape(NW, NB_N, EB)
    u_p = jnp.concatenate(
        [interact_user.astype(_i32), jnp.full((NNZ_P - NNZ,), N_USERS, _i32)])
    u3 = u_p.reshape(NW, NB_N, EB)
    u2 = u_p.reshape(NROW, CH)
    zeros = jnp.zeros((N_ACC, CH), _f32)
    ones_row = jnp.zeros((CH,), _f32).at[0].set(1.0)
    pad_rows = jnp.zeros((N_ACC - N_ENT, CH), _f32)

    ee = jnp.concatenate([entity_emb, pad_rows])
    ent_res = jnp.concatenate([entity_emb, pad_rows])
    usr_res = jnp.concatenate([user_emb, pad_rows])

    cnt_p = _sc_cnt(head3, zeros, ones_row)
    for _ in range(N_HOPS):
        rows_p = _sc_edge(ee, tail3, head3, zeros)
        entity_agg, ee, ent_res = _tc_entity(rows_p, cnt_p, ent_res)
        r_rows, um_p = _sc_gather_scatter(entity_agg, i3, u3, zeros)
        user_mean = _tc_combine(um_p)
        m_rows = _sc_gather(user_mean, u3)
        r3 = r_rows.reshape(NROW, CH, CH)
        s2 = _tc_score(r3, m_rows.reshape(NROW, CH, CH))
        soft2 = _tc_softmax(s2, u2)
        w3 = _tc_wmul(soft2, r3)
        ua_p = _sc_scatter(w3.reshape(NNZ_P, CH), u3, zeros)
        usr_res = _tc_user(ua_p, usr_res)

    return (usr_res[:N_USERS], ent_res[:N_ENT])


# speedup vs baseline: 1.0447x; 1.0341x over previous
import functools

import jax
import jax.numpy as jnp
from jax import lax
from jax.experimental import pallas as pl
from jax.experimental.pallas import tpu as pltpu
from jax.experimental.pallas import tpu_sc as plsc

N_USERS = 10000
N_ENT = 10000
N_ITEMS = 5000
CH = 128
E = 320000
NNZ = 200000
TEMP = 0.2
N_HOPS = 2

NC = 2
NS = 16
EB = 128
E_P = 327680
NB_E = E_P // EB // (NC * NS)
NNZ_P = 200704
NB_N = NNZ_P // EB // (NC * NS)
N_ACC = 10240
ZR = N_ACC // NS
NROW = NNZ_P // CH

_MESH = plsc.VectorSubcoreMesh(core_axis_name="c", subcore_axis_name="s")
_f32 = jnp.float32
_i32 = jnp.int32


@functools.partial(
    pl.kernel, mesh=_MESH,
    out_type=jax.ShapeDtypeStruct((NC, N_ACC, CH), _f32),
    scratch_types=[
        pltpu.VMEM((EB,), _i32),
        pltpu.VMEM((EB,), _i32),
        pltpu.VMEM((EB, CH), _f32),
        pltpu.VMEM_SHARED((N_ACC, CH), _f32),
        pltpu.SemaphoreType.DMA,
    ])
def _sc_edge(ee_hbm, tail_hbm, head_hbm, zeros_hbm,
             out_rows, tail_v, head_v, rows_v, acc, sem):
    c = lax.axis_index("c")
    s = lax.axis_index("s")
    w = s * NC + c

    @pl.when(s == 0)
    def _():
        pltpu.sync_copy(zeros_hbm, acc)

    plsc.subcore_barrier()

    def body(i, carry):
        off = (w * NB_E + i) * EB
        pltpu.sync_copy(tail_hbm.at[pl.ds(off, EB)], tail_v)
        pltpu.sync_copy(head_hbm.at[pl.ds(off, EB)], head_v)
        pltpu.async_copy(ee_hbm.at[tail_v], rows_v, sem).wait()
        pltpu.sync_copy(rows_v, acc.at[head_v], add=True)
        return carry

    lax.fori_loop(0, NB_E, body, 0)
    plsc.subcore_barrier()

    @pl.when(s == 0)
    def _():
        pltpu.sync_copy(acc, out_rows.at[c])


@functools.partial(
    pl.kernel, mesh=_MESH,
    out_type=jax.ShapeDtypeStruct((NC, N_ACC, CH), _f32),
    scratch_types=[
        pltpu.VMEM((EB,), _i32),
        pltpu.VMEM((EB, CH), _f32),
        pltpu.VMEM_SHARED((N_ACC, CH), _f32),
    ])
def _sc_cnt(head_hbm, zeros_hbm, ones_hbm, out_cnt,
            head_v, ones_v, cacc):
    c = lax.axis_index("c")
    s = lax.axis_index("s")
    w = s * NC + c

    @pl.when(s == 0)
    def _():
        pltpu.sync_copy(zeros_hbm, cacc)

    def fill(i, carry):
        pltpu.sync_copy(ones_hbm, ones_v.at[i])
        return carry

    lax.fori_loop(0, EB, fill, 0)
    plsc.subcore_barrier()

    def body(i, carry):
        off = (w * NB_E + i) * EB
        pltpu.sync_copy(head_hbm.at[pl.ds(off, EB)], head_v)
        pltpu.sync_copy(ones_v, cacc.at[head_v], add=True)
        return carry

    lax.fori_loop(0, NB_E, body, 0)
    plsc.subcore_barrier()

    @pl.when(s == 0)
    def _():
        pltpu.sync_copy(cacc, out_cnt.at[c])


@functools.partial(
    pl.kernel, mesh=_MESH,
    out_type=[jax.ShapeDtypeStruct((NNZ_P, CH), _f32),
              jax.ShapeDtypeStruct((NC, N_ACC, CH), _f32)],
    scratch_types=[
        pltpu.VMEM((EB,), _i32),
        pltpu.VMEM((EB,), _i32),
        pltpu.VMEM((EB, CH), _f32),
        pltpu.VMEM_SHARED((N_ACC, CH), _f32),
        pltpu.SemaphoreType.DMA,
    ])
def _sc_gather_scatter(tab_hbm, idx_hbm, u_hbm, zeros_hbm,
                       out_rows, out_part, idx_v, u_v, rows_v, acc, sem):
    c = lax.axis_index("c")
    s = lax.axis_index("s")
    w = s * NC + c

    @pl.when(s == 0)
    def _():
        pltpu.sync_copy(zeros_hbm, acc)

    plsc.subcore_barrier()

    def body(i, carry):
        off = (w * NB_N + i) * EB
        pltpu.sync_copy(idx_hbm.at[pl.ds(off, EB)], idx_v)
        pltpu.sync_copy(u_hbm.at[pl.ds(off, EB)], u_v)
        pltpu.async_copy(tab_hbm.at[idx_v], rows_v, sem).wait()
        pltpu.sync_copy(rows_v, out_rows.at[pl.ds(off, EB)])
        pltpu.sync_copy(rows_v, acc.at[u_v], add=True)
        return carry

    lax.fori_loop(0, NB_N, body, 0)
    plsc.subcore_barrier()

    @pl.when(s == 0)
    def _():
        pltpu.sync_copy(acc, out_part.at[c])


@functools.partial(
    pl.kernel, mesh=_MESH,
    out_type=jax.ShapeDtypeStruct((NNZ_P, CH), _f32),
    scratch_types=[
        pltpu.VMEM((EB,), _i32),
        pltpu.VMEM((EB, CH), _f32),
        pltpu.SemaphoreType.DMA,
    ])
def _sc_gather(tab_hbm, u_hbm, out_rows, u_v, rows_v, sem):
    c = lax.axis_index("c")
    s = lax.axis_index("s")
    w = s * NC + c

    def body(i, carry):
        off = (w * NB_N + i) * EB
        pltpu.sync_copy(u_hbm.at[pl.ds(off, EB)], u_v)
        pltpu.async_copy(tab_hbm.at[u_v], rows_v, sem).wait()
        pltpu.sync_copy(rows_v, out_rows.at[pl.ds(off, EB)])
        return carry

    lax.fori_loop(0, NB_N, body, 0)


@functools.partial(
    pl.kernel, mesh=_MESH,
    out_type=jax.ShapeDtypeStruct((NC, N_ACC, CH), _f32),
    scratch_types=[
        pltpu.VMEM((EB,), _i32),
        pltpu.VMEM((EB, CH), _f32),
        pltpu.VMEM_SHARED((N_ACC, CH), _f32),
    ])
def _sc_scatter(w_hbm, u_hbm, zeros_hbm, out_part, u_v, rows_v, acc):
    c = lax.axis_index("c")
    s = lax.axis_index("s")
    w = s * NC + c

    @pl.when(s == 0)
    def _():
        pltpu.sync_copy(zeros_hbm, acc)

    plsc.subcore_barrier()

    def body(i, carry):
        off = (w * NB_N + i) * EB
        pltpu.sync_copy(u_hbm.at[pl.ds(off, EB)], u_v)
        pltpu.sync_copy(w_hbm.at[pl.ds(off, EB)], rows_v)
        pltpu.sync_copy(rows_v, acc.at[u_v], add=True)
        return carry

    lax.fori_loop(0, NB_N, body, 0)
    plsc.subcore_barrier()

    @pl.when(s == 0)
    def _():
        pltpu.sync_copy(acc, out_part.at[c])


_BR = 512
_NBLK = N_ACC // _BR


def _tc_entity_body(p_ref, c_ref, res_ref, agg_ref, ee_ref, out_ref):
    sums = p_ref[0] + p_ref[1]
    cnt = jnp.maximum(c_ref[0, :, 0:1] + c_ref[1, :, 0:1], 1.0)
    agg = sums / cnt
    ss = jnp.sum(agg * agg, axis=1, keepdims=True)
    nn = agg * lax.rsqrt(jnp.maximum(ss, 1e-24))
    agg_ref[...] = agg
    ee_ref[...] = nn
    out_ref[...] = res_ref[...] + nn


def _tc_entity(p, cnt_p, res):
    return pl.pallas_call(
        _tc_entity_body,
        grid=(_NBLK,),
        in_specs=[
            pl.BlockSpec((NC, _BR, CH), lambda i: (0, i, 0)),
            pl.BlockSpec((NC, _BR, CH), lambda i: (0, i, 0)),
            pl.BlockSpec((_BR, CH), lambda i: (i, 0)),
        ],
        out_specs=[
            pl.BlockSpec((_BR, CH), lambda i: (i, 0)),
            pl.BlockSpec((_BR, CH), lambda i: (i, 0)),
            pl.BlockSpec((_BR, CH), lambda i: (i, 0)),
        ],
        out_shape=[
            jax.ShapeDtypeStruct((N_ACC, CH), _f32),
            jax.ShapeDtypeStruct((N_ACC, CH), _f32),
            jax.ShapeDtypeStruct((N_ACC, CH), _f32),
        ],
    )(p, cnt_p, res)


def _tc_user_body(p_ref, res_ref, out_ref):
    agg = p_ref[0] + p_ref[1]
    ss = jnp.sum(agg * agg, axis=1, keepdims=True)
    nn = agg * lax.rsqrt(jnp.maximum(ss, 1e-24))
    out_ref[...] = res_ref[...] + nn


def _tc_user(p, res):
    return pl.pallas_call(
        _tc_user_body,
        grid=(_NBLK,),
        in_specs=[
            pl.BlockSpec((NC, _BR, CH), lambda i: (0, i, 0)),
            pl.BlockSpec((_BR, CH), lambda i: (i, 0)),
        ],
        out_specs=pl.BlockSpec((_BR, CH), lambda i: (i, 0)),
        out_shape=jax.ShapeDtypeStruct((N_ACC, CH), _f32),
    )(p, res)


def _tc_combine_body(p_ref, out_ref):
    out_ref[...] = p_ref[0] + p_ref[1]


def _tc_combine(p):
    return pl.pallas_call(
        _tc_combine_body,
        grid=(_NBLK,),
        in_specs=[pl.BlockSpec((NC, _BR, CH), lambda i: (0, i, 0))],
        out_specs=pl.BlockSpec((_BR, CH), lambda i: (i, 0)),
        out_shape=jax.ShapeDtypeStruct((N_ACC, CH), _f32),
    )(p)


_SB = 8


def _tc_score_body(r_ref, m_ref, s_ref):
    d = r_ref[...] - m_ref[...] + 1e-6
    s_ref[...] = jnp.sum(d * d, axis=2)


def _tc_score(r3, m3):
    return pl.pallas_call(
        _tc_score_body,
        grid=(NROW // _SB,),
        in_specs=[
            pl.BlockSpec((_SB, CH, CH), lambda i: (i, 0, 0)),
            pl.BlockSpec((_SB, CH, CH), lambda i: (i, 0, 0)),
        ],
        out_specs=pl.BlockSpec((_SB, CH), lambda i: (i, 0)),
        out_shape=jax.ShapeDtypeStruct((NROW, CH), _f32),
    )(r3, m3)


_KSTEPS = (1, 2, 4, 8, 16, 32, 64, 128, 256, 512, 1024)


def _shift_dn(x, k, fill):
    rr, cc = x.shape
    if k % cc == 0:
        r = k // cc
        top = jnp.full((r, cc), fill, x.dtype)
        return jnp.concatenate([top, x[:-r]], axis=0)
    xprev = jnp.concatenate(
        [jnp.full((1, cc), fill, x.dtype), x[:-1]], axis=0)
    return jnp.concatenate([xprev[:, cc - k:], x[:, :cc - k]], axis=1)


def _shift_up(x, k, fill):
    rr, cc = x.shape
    if k % cc == 0:
        r = k // cc
        bot = jnp.full((r, cc), fill, x.dtype)
        return jnp.concatenate([x[r:], bot], axis=0)
    xnext = jnp.concatenate(
        [x[1:], jnp.full((1, cc), fill, x.dtype)], axis=0)
    return jnp.concatenate([x[:, k:], xnext[:, :k]], axis=1)


def _tc_softmax_body(s2_ref, u_ref, soft_ref):
    ss = s2_ref[...]
    u = u_ref[...]
    s = jnp.sqrt(ss) * (1.0 / TEMP)
    m = s
    for k in _KSTEPS:
        us = _shift_dn(u, k, -1)
        ms = _shift_dn(m, k, 0.0)
        m = jnp.where(us == u, jnp.maximum(m, ms), m)
    mb = s
    for k in _KSTEPS:
        us = _shift_up(u, k, -1)
        ms = _shift_up(mb, k, 0.0)
        mb = jnp.where(us == u, jnp.maximum(mb, ms), mb)
    mm = jnp.maximum(m, mb)
    e = jnp.exp(s - mm)
    lf = e
    for k in _KSTEPS:
        us = _shift_dn(u, k, -1)
        ls = _shift_dn(lf, k, 0.0)
        lf = lf + jnp.where(us == u, ls, 0.0)
    lb = e
    for k in _KSTEPS:
        us = _shift_up(u, k, -1)
        ls = _shift_up(lb, k, 0.0)
        lb = lb + jnp.where(us == u, ls, 0.0)
    denom = lf + lb - e
    soft_ref[...] = e / denom


def _tc_softmax(s2, u2):
    return pl.pallas_call(
        _tc_softmax_body,
        out_shape=jax.ShapeDtypeStruct((NROW, CH), _f32),
    )(s2, u2)


def _tc_wmul_body(soft_ref, r_ref, w_ref):
    w_ref[...] = r_ref[...] * soft_ref[...][:, :, None]


def _tc_wmul(soft2, r3):
    return pl.pallas_call(
        _tc_wmul_body,
        grid=(NROW // _SB,),
        in_specs=[
            pl.BlockSpec((_SB, CH), lambda i: (i, 0)),
            pl.BlockSpec((_SB, CH, CH), lambda i: (i, 0, 0)),
        ],
        out_specs=pl.BlockSpec((_SB, CH, CH), lambda i: (i, 0, 0)),
        out_shape=jax.ShapeDtypeStruct((NROW, CH, CH), _f32),
    )(soft2, r3)


def kernel(user_emb, entity_emb, weight, interact_values, edge_index,
           edge_type, interact_user, interact_item):
    tail_p = jnp.concatenate(
        [edge_index[1].astype(_i32), jnp.zeros((E_P - E,), _i32)])
    head_p = jnp.concatenate(
        [edge_index[0].astype(_i32), jnp.full((E_P - E,), N_ENT, _i32)])
    i_p = jnp.concatenate(
        [interact_item.astype(_i32), jnp.zeros((NNZ_P - NNZ,), _i32)])
    u_p = jnp.concatenate(
        [interact_user.astype(_i32), jnp.full((NNZ_P - NNZ,), N_USERS, _i32)])
    u2 = u_p.reshape(NROW, CH)
    zeros = jnp.zeros((N_ACC, CH), _f32)
    ones_row = jnp.zeros((CH,), _f32).at[0].set(1.0)
    pad_rows = jnp.zeros((N_ACC - N_ENT, CH), _f32)

    ee = jnp.concatenate([entity_emb, pad_rows])
    ent_res = jnp.concatenate([entity_emb, pad_rows])
    usr_res = jnp.concatenate([user_emb, pad_rows])

    _USE_EDGE, _USE_GS, _USE_G, _USE_SC = True, True, True, True
    cnt_p = _sc_cnt(head_p, zeros, ones_row)
    for _ in range(N_HOPS):
        if _USE_EDGE:
            rows_p = _sc_edge(ee, tail_p, head_p, zeros)
        else:
            rows_p = jnp.stack([
                jax.ops.segment_sum(ee[tail_p], head_p, num_segments=N_ACC),
                jnp.zeros((N_ACC, CH), _f32)])
        entity_agg, ee, ent_res = _tc_entity(rows_p, cnt_p, ent_res)
        if _USE_GS:
            r_rows, um_p = _sc_gather_scatter(entity_agg, i_p, u_p, zeros)
        else:
            r_rows = entity_agg[i_p]
            um_p = jnp.stack([
                jax.ops.segment_sum(r_rows, u_p, num_segments=N_ACC),
                jnp.zeros((N_ACC, CH), _f32)])
        user_mean = _tc_combine(um_p)
        if _USE_G:
            m_rows = _sc_gather(user_mean, u_p)
        else:
            m_rows = user_mean[u_p]
        r3 = r_rows.reshape(NROW, CH, CH)
        s2 = _tc_score(r3, m_rows.reshape(NROW, CH, CH))
        soft2 = _tc_softmax(s2, u2)
        w3 = _tc_wmul(soft2, r3)
        if _USE_SC:
            ua_p = _sc_scatter(w3.reshape(NNZ_P, CH), u_p, zeros)
        else:
            ua_p = jnp.stack([
                jax.ops.segment_sum(w3.reshape(NNZ_P, CH), u_p,
                                    num_segments=N_ACC),
                jnp.zeros((N_ACC, CH), _f32)])
        usr_res = _tc_user(ua_p, usr_res)

    return (usr_res[:N_USERS], ent_res[:N_ENT])
